# baseline (speedup 1.0000x reference)
import jax
import jax.numpy as jnp
from jax import lax
from jax.experimental import pallas as pl
from jax.experimental.pallas import tpu as pltpu
from jax.experimental.pallas import tpu_sc as plsc

N_NODES = 10000
N_EDGES = 320000
D = 128
N_RELS = 10

NC = 2
NS = 16
L = 16
NW = NC * NS

B = 80
PER_W = N_EDGES // NW
N_CHUNKS = PER_W // B
NBUF = 2


def _sc_body(h_hbm, src_hbm, dst_hbm, et_hbm, w_hbm, out_hbm,
             h_sp, idx_u, idx_v, iet, out_c, rows_u, rows_v, w_v,
             sus, svs):
    sid = lax.axis_index("s")
    wid = sid * NC + lax.axis_index("c")
    base0 = wid * PER_W

    stripe = 640

    @pl.when(sid < NS - 1)
    def _stage_full():
        pltpu.sync_copy(h_hbm.at[pl.ds(sid * stripe, stripe)],
                        h_sp.at[pl.ds(sid * stripe, stripe)])

    @pl.when(sid == NS - 1)
    def _stage_tail():
        tail = N_NODES - (NS - 1) * stripe
        pltpu.sync_copy(h_hbm.at[pl.ds((NS - 1) * stripe, tail)],
                        h_sp.at[pl.ds((NS - 1) * stripe, tail)])

    pltpu.sync_copy(w_hbm, w_v)
    plsc.subcore_barrier()

    lanes = lax.iota(jnp.int32, L)

    def fire(c, b):
        base = base0 + c * B
        pltpu.sync_copy(src_hbm.at[pl.ds(base, B)], idx_u.at[b])
        pltpu.sync_copy(dst_hbm.at[pl.ds(base, B)], idx_v.at[b])
        pltpu.async_copy(h_sp.at[idx_u.at[b]], rows_u.at[b], sus.at[b])
        pltpu.async_copy(h_sp.at[idx_v.at[b]], rows_v.at[b], svs.at[b])

    def wait(c, b):
        pltpu.make_async_copy(h_sp.at[idx_u.at[b]], rows_u.at[b], sus.at[b]).wait()
        pltpu.make_async_copy(h_sp.at[idx_v.at[b]], rows_v.at[b], svs.at[b]).wait()

    def compute(c, b):
        ru = rows_u.at[b]
        rv = rows_v.at[b]
        pltpu.sync_copy(et_hbm.at[pl.ds(base0 + c * B, B)], iet)

        def group_body(g, _):
            eid = lanes + g * L
            r = iet[pl.ds(g * L, L)]
            acc = jnp.zeros((L,), jnp.float32)
            for d in range(D):
                col = jnp.full((L,), d, jnp.int32)
                u = plsc.load_gather(ru, [eid, col])
                v = plsc.load_gather(rv, [eid, col])
                w = plsc.load_gather(w_v, [r, col])
                acc = acc + (u * v) * w
            out_c[pl.ds(g * L, L)] = 1.0 / (1.0 + jnp.exp(-acc))
            return _

        lax.fori_loop(0, B // L, group_body, None)
        pltpu.sync_copy(out_c, out_hbm.at[pl.ds(base0 + c * B, B)])

    for b in range(NBUF):
        fire(b, b)

    def ring_body(cg, _):
        for b in range(NBUF):
            c = cg * NBUF + b
            wait(c, b)
            compute(c, b)

            @pl.when(c + NBUF < N_CHUNKS)
            def _f():
                fire(c + NBUF, b)
        return _

    lax.fori_loop(0, (N_CHUNKS - 1) // NBUF, ring_body, None)
    wait(N_CHUNKS - 1, (N_CHUNKS - 1) % NBUF)
    compute(N_CHUNKS - 1, (N_CHUNKS - 1) % NBUF)


@jax.jit
def _dist_mult_sc(h, src, dst, et, W):
    mesh = plsc.VectorSubcoreMesh(core_axis_name="c", subcore_axis_name="s",
                                  num_cores=NC, num_subcores=NS)
    return pl.kernel(
        _sc_body,
        out_type=jax.ShapeDtypeStruct((N_EDGES,), jnp.float32),
        mesh=mesh,
        scratch_types=[
            pltpu.VMEM_SHARED((N_NODES, D), jnp.float32),
            pltpu.VMEM((NBUF, B), jnp.int32),
            pltpu.VMEM((NBUF, B), jnp.int32),
            pltpu.VMEM((B,), jnp.int32),
            pltpu.VMEM((B,), jnp.float32),
            pltpu.VMEM((NBUF, B, D), jnp.float32),
            pltpu.VMEM((NBUF, B, D), jnp.float32),
            pltpu.VMEM((N_RELS, D), jnp.float32),
            pltpu.SemaphoreType.DMA((NBUF,)),
            pltpu.SemaphoreType.DMA((NBUF,)),
        ],
        compiler_params=pltpu.CompilerParams(needs_layout_passes=False),
    )(h, src, dst, et, W)


def kernel(h, edge_index, edge_type, W):
    src = edge_index[0].astype(jnp.int32)
    dst = edge_index[1].astype(jnp.int32)
    et = edge_type.astype(jnp.int32)
    return _dist_mult_sc(h, src, dst, et, W)


# speedup vs baseline: 1.3256x; 1.3256x over previous
import jax
import jax.numpy as jnp
from jax import lax
from jax.experimental import pallas as pl
from jax.experimental.pallas import tpu as pltpu
from jax.experimental.pallas import tpu_sc as plsc

N_NODES = 10000
N_EDGES = 320000
D = 128
N_RELS = 10

NC = 2
NS = 16
L = 16
NW = NC * NS

B = 80
PER_W = N_EDGES // NW
N_CHUNKS = PER_W // B
NBUF = 3

_DNUMS = lax.GatherDimensionNumbers(
    offset_dims=(), collapsed_slice_dims=(0,), start_index_map=(0,))


def _permute(x, idx):
    return lax.gather(x, idx[:, None], _DNUMS, slice_sizes=(1,),
                      mode=lax.GatherScatterMode.PROMISE_IN_BOUNDS)


def _sc_body(h_hbm, src_hbm, dst_hbm, et_hbm, w_hbm, out_hbm,
             idx_u, idx_v, idx_w, out_c, rows_u, rows_v, rows_w,
             sus, svs, sws):
    sid = lax.axis_index("s")
    wid = sid * NC + lax.axis_index("c")
    base0 = wid * PER_W

    lanes = lax.iota(jnp.int32, L)
    perms = [lanes ^ (1 << k) for k in range(4)]

    def hsum(x):
        for p in perms:
            x = x + _permute(x, p)
        return x

    def fire(c, b):
        base = base0 + c * B
        pltpu.sync_copy(src_hbm.at[pl.ds(base, B)], idx_u.at[b])
        pltpu.sync_copy(dst_hbm.at[pl.ds(base, B)], idx_v.at[b])
        pltpu.sync_copy(et_hbm.at[pl.ds(base, B)], idx_w.at[b])
        pltpu.async_copy(h_hbm.at[idx_u.at[b]], rows_u.at[b], sus.at[b])
        pltpu.async_copy(h_hbm.at[idx_v.at[b]], rows_v.at[b], svs.at[b])
        pltpu.async_copy(w_hbm.at[idx_w.at[b]], rows_w.at[b], sws.at[b])

    def wait(c, b):
        pltpu.make_async_copy(h_hbm.at[idx_u.at[b]], rows_u.at[b], sus.at[b]).wait()
        pltpu.make_async_copy(h_hbm.at[idx_v.at[b]], rows_v.at[b], svs.at[b]).wait()
        pltpu.make_async_copy(w_hbm.at[idx_w.at[b]], rows_w.at[b], sws.at[b]).wait()

    def compute(c, b):
        ru = rows_u.at[b]
        rv = rows_v.at[b]
        rw = rows_w.at[b]

        def group_body(g, _):
            def edge_body(e16, out_vec):
                e = g * L + e16
                acc = None
                for j in range(D // L):
                    u = ru[e, pl.ds(j * L, L)]
                    v = rv[e, pl.ds(j * L, L)]
                    w = rw[e, pl.ds(j * L, L)]
                    p = (u * v) * w
                    acc = p if acc is None else acc + p
                s = hsum(acc)
                return jnp.where(lanes == e16, s, out_vec)

            out_vec = lax.fori_loop(0, L, edge_body, jnp.zeros((L,), jnp.float32))
            out_c[pl.ds(g * L, L)] = 1.0 / (1.0 + jnp.exp(-out_vec))
            return _

        lax.fori_loop(0, B // L, group_body, None)
        pltpu.sync_copy(out_c, out_hbm.at[pl.ds(base0 + c * B, B)])

    for b in range(NBUF):
        fire(b, b)

    def ring_body(cg, _):
        for b in range(NBUF):
            c = cg * NBUF + b
            wait(c, b)
            compute(c, b)

            @pl.when(c + NBUF < N_CHUNKS)
            def _f():
                fire(c + NBUF, b)
        return _

    n_full = (N_CHUNKS - 1) // NBUF
    lax.fori_loop(0, n_full, ring_body, None)
    for t in range(n_full * NBUF, N_CHUNKS):
        wait(t, t % NBUF)
        compute(t, t % NBUF)


@jax.jit
def _dist_mult_sc(h, src, dst, et, W):
    mesh = plsc.VectorSubcoreMesh(core_axis_name="c", subcore_axis_name="s",
                                  num_cores=NC, num_subcores=NS)
    return pl.kernel(
        _sc_body,
        out_type=jax.ShapeDtypeStruct((N_EDGES,), jnp.float32),
        mesh=mesh,
        scratch_types=[
            pltpu.VMEM((NBUF, B), jnp.int32),
            pltpu.VMEM((NBUF, B), jnp.int32),
            pltpu.VMEM((NBUF, B), jnp.int32),
            pltpu.VMEM((B,), jnp.float32),
            pltpu.VMEM((NBUF, B, D), jnp.float32),
            pltpu.VMEM((NBUF, B, D), jnp.float32),
            pltpu.VMEM((NBUF, B, D), jnp.float32),
            pltpu.SemaphoreType.DMA((NBUF,)),
            pltpu.SemaphoreType.DMA((NBUF,)),
            pltpu.SemaphoreType.DMA((NBUF,)),
        ],
        compiler_params=pltpu.CompilerParams(needs_layout_passes=False),
    )(h, src, dst, et, W)


def kernel(h, edge_index, edge_type, W):
    src = edge_index[0].astype(jnp.int32)
    dst = edge_index[1].astype(jnp.int32)
    et = edge_type.astype(jnp.int32)
    return _dist_mult_sc(h, src, dst, et, W)


# speedup vs baseline: 1.3271x; 1.0011x over previous
import jax
import jax.numpy as jnp
from jax import lax
from jax.experimental import pallas as pl
from jax.experimental.pallas import tpu as pltpu
from jax.experimental.pallas import tpu_sc as plsc

N_NODES = 10000
N_EDGES = 320000
D = 128
N_RELS = 10

NC = 2
NS = 16
L = 16
NW = NC * NS

B = 80
PER_W = N_EDGES // NW
N_CHUNKS = PER_W // B
NBUF = 3

_DNUMS = lax.GatherDimensionNumbers(
    offset_dims=(), collapsed_slice_dims=(0,), start_index_map=(0,))


def _permute(x, idx):
    return lax.gather(x, idx[:, None], _DNUMS, slice_sizes=(1,),
                      mode=lax.GatherScatterMode.PROMISE_IN_BOUNDS)


def _sc_body(h_hbm, src_hbm, dst_hbm, et_hbm, w_hbm, out_hbm,
             idx_u, idx_v, idx_w, out_c, rows_u, rows_v, rows_w,
             sus, svs, sws, sis, sos):
    sid = lax.axis_index("s")
    wid = sid * NC + lax.axis_index("c")
    base0 = wid * PER_W

    lanes = lax.iota(jnp.int32, L)
    perms = [lanes ^ (1 << k) for k in range(4)]

    def hsum(x):
        for p in perms:
            x = x + _permute(x, p)
        return x

    def fire_idx(c, b):
        base = base0 + c * B
        pltpu.async_copy(src_hbm.at[pl.ds(base, B)], idx_u.at[b], sis.at[b])
        pltpu.async_copy(dst_hbm.at[pl.ds(base, B)], idx_v.at[b], sis.at[b])
        pltpu.async_copy(et_hbm.at[pl.ds(base, B)], idx_w.at[b], sis.at[b])

    def wait_idx(c, b):
        for ref in (idx_u, idx_v, idx_w):
            pltpu.make_async_copy(src_hbm.at[pl.ds(base0, B)],
                                  ref.at[b], sis.at[b]).wait()

    def fire_rows(c, b):
        pltpu.async_copy(h_hbm.at[idx_u.at[b]], rows_u.at[b], sus.at[b])
        pltpu.async_copy(h_hbm.at[idx_v.at[b]], rows_v.at[b], svs.at[b])
        pltpu.async_copy(w_hbm.at[idx_w.at[b]], rows_w.at[b], sws.at[b])

    def wait_rows(c, b):
        pltpu.make_async_copy(h_hbm.at[idx_u.at[b]], rows_u.at[b], sus.at[b]).wait()
        pltpu.make_async_copy(h_hbm.at[idx_v.at[b]], rows_v.at[b], svs.at[b]).wait()
        pltpu.make_async_copy(w_hbm.at[idx_w.at[b]], rows_w.at[b], sws.at[b]).wait()

    def wait_out(ob):
        pltpu.make_async_copy(out_c.at[ob], out_hbm.at[pl.ds(base0, B)],
                              sos.at[ob]).wait()

    def compute(c, b):
        ru = rows_u.at[b]
        rv = rows_v.at[b]
        rw = rows_w.at[b]
        ob = c % 2

        @pl.when(c >= 2)
        def _wo():
            wait_out(ob)

        def group_body(g, _):
            def edge_body(e16, out_vec):
                e = g * L + e16
                acc = None
                for j in range(D // L):
                    u = ru[e, pl.ds(j * L, L)]
                    v = rv[e, pl.ds(j * L, L)]
                    w = rw[e, pl.ds(j * L, L)]
                    p = (u * v) * w
                    acc = p if acc is None else acc + p
                s = hsum(acc)
                return jnp.where(lanes == e16, s, out_vec)

            out_vec = lax.fori_loop(0, L, edge_body, jnp.zeros((L,), jnp.float32))
            out_c[ob, pl.ds(g * L, L)] = 1.0 / (1.0 + jnp.exp(-out_vec))
            return _

        lax.fori_loop(0, B // L, group_body, None)
        pltpu.async_copy(out_c.at[ob], out_hbm.at[pl.ds(base0 + c * B, B)],
                         sos.at[ob])

    for b in range(NBUF):
        fire_idx(b, b)
    wait_idx(0, 0)
    fire_rows(0, 0)
    wait_idx(1, 1)
    fire_rows(1, 1)

    def ring_body(cg, _):
        for b in range(NBUF):
            c = cg * NBUF + b
            wait_rows(c, b)

            @pl.when(c + 2 < N_CHUNKS)
            def _fr():
                b2 = (c + 2) % NBUF
                wait_idx(c + 2, b2)
                fire_rows(c + 2, b2)

            @pl.when(c + NBUF < N_CHUNKS)
            def _fi():
                fire_idx(c + NBUF, b)

            compute(c, b)
        return _

    n_full = N_CHUNKS // NBUF
    lax.fori_loop(0, n_full, ring_body, None)
    for t in range(n_full * NBUF, N_CHUNKS):
        wait_rows(t, t % NBUF)
        compute(t, t % NBUF)
    wait_out((N_CHUNKS - 2) % 2)
    wait_out((N_CHUNKS - 1) % 2)


@jax.jit
def _dist_mult_sc(h, src, dst, et, W):
    mesh = plsc.VectorSubcoreMesh(core_axis_name="c", subcore_axis_name="s",
                                  num_cores=NC, num_subcores=NS)
    return pl.kernel(
        _sc_body,
        out_type=jax.ShapeDtypeStruct((N_EDGES,), jnp.float32),
        mesh=mesh,
        scratch_types=[
            pltpu.VMEM((NBUF, B), jnp.int32),
            pltpu.VMEM((NBUF, B), jnp.int32),
            pltpu.VMEM((NBUF, B), jnp.int32),
            pltpu.VMEM((2, B), jnp.float32),
            pltpu.VMEM((NBUF, B, D), jnp.float32),
            pltpu.VMEM((NBUF, B, D), jnp.float32),
            pltpu.VMEM((NBUF, B, D), jnp.float32),
            pltpu.SemaphoreType.DMA((NBUF,)),
            pltpu.SemaphoreType.DMA((NBUF,)),
            pltpu.SemaphoreType.DMA((NBUF,)),
            pltpu.SemaphoreType.DMA((NBUF,)),
            pltpu.SemaphoreType.DMA((2,)),
        ],
        compiler_params=pltpu.CompilerParams(needs_layout_passes=False),
    )(h, src, dst, et, W)


def kernel(h, edge_index, edge_type, W):
    src = edge_index[0].astype(jnp.int32)
    dst = edge_index[1].astype(jnp.int32)
    et = edge_type.astype(jnp.int32)
    return _dist_mult_sc(h, src, dst, et, W)


# speedup vs baseline: 7.4574x; 5.6195x over previous
import jax
import jax.numpy as jnp
from jax import lax
from jax.experimental import pallas as pl
from jax.experimental.pallas import tpu as pltpu
from jax.experimental.pallas import tpu_sc as plsc

N_NODES = 10000
N_EDGES = 320000
D = 128
N_RELS = 10

NC = 2
NS = 16
L = 16
NW = NC * NS

B = 80
PER_W = N_EDGES // NW
N_CHUNKS = PER_W // B
NBUF = 2

_DNUMS = lax.GatherDimensionNumbers(
    offset_dims=(), collapsed_slice_dims=(0,), start_index_map=(0,))


def _permute(x, idx):
    return lax.gather(x, idx[:, None], _DNUMS, slice_sizes=(1,),
                      mode=lax.GatherScatterMode.PROMISE_IN_BOUNDS)


def _sc_body(h_hbm, src_hbm, dst_hbm, et_hbm, w_hbm, out_hbm,
             h_sp, idx_u, idx_v, idx_w, out_c, rows_u, rows_v, w_v,
             sus, svs, sis, sos):
    sid = lax.axis_index("s")
    wid = sid * NC + lax.axis_index("c")
    base0 = wid * PER_W

    stripe = 640

    @pl.when(sid < NS - 1)
    def _stage_full():
        pltpu.sync_copy(h_hbm.at[pl.ds(sid * stripe, stripe)],
                        h_sp.at[pl.ds(sid * stripe, stripe)])

    @pl.when(sid == NS - 1)
    def _stage_tail():
        tail = N_NODES - (NS - 1) * stripe
        pltpu.sync_copy(h_hbm.at[pl.ds((NS - 1) * stripe, tail)],
                        h_sp.at[pl.ds((NS - 1) * stripe, tail)])

    pltpu.sync_copy(w_hbm, w_v)
    plsc.subcore_barrier()

    lanes = lax.iota(jnp.int32, L)
    perms = [lanes ^ (1 << k) for k in range(4)]

    def hsum(x):
        for p in perms:
            x = x + _permute(x, p)
        return x

    def fire_idx(c, b):
        base = base0 + c * B
        pltpu.async_copy(src_hbm.at[pl.ds(base, B)], idx_u.at[b], sis.at[b])
        pltpu.async_copy(dst_hbm.at[pl.ds(base, B)], idx_v.at[b], sis.at[b])
        pltpu.async_copy(et_hbm.at[pl.ds(base, B)], idx_w.at[b], sis.at[b])

    def wait_idx(c, b):
        for ref in (idx_u, idx_v, idx_w):
            pltpu.make_async_copy(src_hbm.at[pl.ds(base0, B)],
                                  ref.at[b], sis.at[b]).wait()

    def fire_rows(c, b):
        pltpu.async_copy(h_sp.at[idx_u.at[b]], rows_u.at[b], sus.at[b])
        pltpu.async_copy(h_sp.at[idx_v.at[b]], rows_v.at[b], svs.at[b])

    def wait_rows(c, b):
        pltpu.make_async_copy(h_sp.at[idx_u.at[b]], rows_u.at[b], sus.at[b]).wait()
        pltpu.make_async_copy(h_sp.at[idx_v.at[b]], rows_v.at[b], svs.at[b]).wait()

    def wait_out(ob):
        pltpu.make_async_copy(out_c.at[ob], out_hbm.at[pl.ds(base0, B)],
                              sos.at[ob]).wait()

    def compute(c, b):
        ru = rows_u.at[b]
        rv = rows_v.at[b]
        ob = c % 2

        @pl.when(c >= 2)
        def _wo():
            wait_out(ob)

        def group_body(g, _):
            rvec = idx_w[b, pl.ds(g * L, L)]

            def edge_body(e16, out_vec):
                e = g * L + e16
                r_splat = _permute(rvec, jnp.full((L,), e16, jnp.int32))
                acc = None
                for j in range(D // L):
                    u = ru[e, pl.ds(j * L, L)]
                    v = rv[e, pl.ds(j * L, L)]
                    w = plsc.load_gather(w_v, [r_splat, lanes + (j * L)])
                    p = (u * v) * w
                    acc = p if acc is None else acc + p
                s = hsum(acc)
                return jnp.where(lanes == e16, s, out_vec)

            out_vec = lax.fori_loop(0, L, edge_body, jnp.zeros((L,), jnp.float32))
            out_c[ob, pl.ds(g * L, L)] = 1.0 / (1.0 + jnp.exp(-out_vec))
            return _

        lax.fori_loop(0, B // L, group_body, None)
        pltpu.async_copy(out_c.at[ob], out_hbm.at[pl.ds(base0 + c * B, B)],
                         sos.at[ob])

    fire_idx(0, 0)
    fire_idx(1, 1)
    wait_idx(0, 0)
    fire_rows(0, 0)

    def ring_body(cg, _):
        for b in range(NBUF):
            c = cg * NBUF + b
            wait_rows(c, b)

            @pl.when(c + 1 < N_CHUNKS)
            def _fr():
                wait_idx(c + 1, 1 - b)
                fire_rows(c + 1, 1 - b)

            compute(c, b)

            @pl.when(c + 2 < N_CHUNKS)
            def _fi():
                fire_idx(c + 2, b)
        return _

    n_full = N_CHUNKS // NBUF
    lax.fori_loop(0, n_full, ring_body, None)
    for t in range(n_full * NBUF, N_CHUNKS):
        wait_rows(t, t % NBUF)
        compute(t, t % NBUF)
    wait_out((N_CHUNKS - 2) % 2)
    wait_out((N_CHUNKS - 1) % 2)


@jax.jit
def _dist_mult_sc(h, src, dst, et, W):
    mesh = plsc.VectorSubcoreMesh(core_axis_name="c", subcore_axis_name="s",
                                  num_cores=NC, num_subcores=NS)
    return pl.kernel(
        _sc_body,
        out_type=jax.ShapeDtypeStruct((N_EDGES,), jnp.float32),
        mesh=mesh,
        scratch_types=[
            pltpu.VMEM_SHARED((N_NODES, D), jnp.float32),
            pltpu.VMEM((NBUF, B), jnp.int32),
            pltpu.VMEM((NBUF, B), jnp.int32),
            pltpu.VMEM((NBUF, B), jnp.int32),
            pltpu.VMEM((2, B), jnp.float32),
            pltpu.VMEM((NBUF, B, D), jnp.float32),
            pltpu.VMEM((NBUF, B, D), jnp.float32),
            pltpu.VMEM((N_RELS, D), jnp.float32),
            pltpu.SemaphoreType.DMA((NBUF,)),
            pltpu.SemaphoreType.DMA((NBUF,)),
            pltpu.SemaphoreType.DMA((NBUF,)),
            pltpu.SemaphoreType.DMA((2,)),
        ],
        compiler_params=pltpu.CompilerParams(needs_layout_passes=False),
    )(h, src, dst, et, W)


def kernel(h, edge_index, edge_type, W):
    src = edge_index[0].astype(jnp.int32)
    dst = edge_index[1].astype(jnp.int32)
    et = edge_type.astype(jnp.int32)
    return _dist_mult_sc(h, src, dst, et, W)
